# baseline (speedup 1.0000x reference)
import functools

import jax
import jax.numpy as jnp
from jax import lax
from jax.experimental import pallas as pl
from jax.experimental.pallas import tpu as pltpu
from jax.experimental.pallas import tpu_sc as plsc

_N = 10000
_E = 320000
_D = 128

_NC = 2
_NS = 16
_LANE = 128
_EPT = _E // (_NC * _NS)
_CHUNKS = -(-_EPT // _LANE)
_EPT_PAD = _CHUNKS * _LANE
_E_PAD = _NC * _NS * _EPT_PAD
_N_PAD = 10240
_NPT = _N_PAD // _NS
_FULL = _EPT // _LANE
_EROWS = _E // _LANE
_TROWS = _EROWS - _FULL * _NC * _NS
_ROWS = _FULL + _TROWS


def _sc_count_body(edges_hbm, out_hbm, idx_v, flags_v, sem):
    cid = lax.axis_index("c")
    sid = lax.axis_index("s")
    wid = cid * _NS + sid

    cp_a = pltpu.async_copy(edges_hbm.at[pl.ds(wid * _FULL, _FULL)],
                            idx_v.at[pl.ds(0, _FULL)], sem)
    cp_b = pltpu.async_copy(edges_hbm.at[pl.ds(_FULL * _NC * _NS, _TROWS)],
                            idx_v.at[pl.ds(_FULL, _TROWS)], sem)

    zeros16 = jnp.zeros((16,), jnp.float32)
    for k in range(_N_PAD // 16):
        flags_v[pl.ds(k * 16, 16)] = zeros16
    cp_a.wait()
    cp_b.wait()

    ones16 = jnp.ones((16,), jnp.float32)
    for j in range(_ROWS):
        for i in range(_LANE // 16):
            idx16 = idx_v[j, 1, pl.ds(i * 16, 16)]
            plsc.store_scatter(flags_v, [idx16], ones16)

    pltpu.sync_copy(flags_v, out_hbm.at[cid, sid])


_sc_count = functools.partial(
    pl.kernel,
    out_type=jax.ShapeDtypeStruct((_NC, _NS, _N_PAD), jnp.float32),
    mesh=plsc.VectorSubcoreMesh(core_axis_name="c", subcore_axis_name="s"),
    compiler_params=pltpu.CompilerParams(needs_layout_passes=False),
    scratch_types=[
        pltpu.VMEM((_ROWS, 2, _LANE), jnp.int32),
        pltpu.VMEM((_N_PAD,), jnp.float32),
        pltpu.SemaphoreType.DMA,
    ],
)(_sc_count_body)


def _tc_mm_body(x_ref, wv_ref, bv_ref, o_ref):
    acc = jnp.dot(x_ref[...], wv_ref[...], preferred_element_type=jnp.float32)
    o_ref[...] = acc + bv_ref[...]


def _tc_mm(x, wv, bv2d):
    blk = 2000
    grid = _N // blk
    return pl.pallas_call(
        _tc_mm_body,
        grid=(grid,),
        in_specs=[
            pl.BlockSpec((blk, _D), lambda j: (j, 0)),
            pl.BlockSpec((_D, _D), lambda j: (0, 0)),
            pl.BlockSpec((1, _D), lambda j: (0, 0)),
        ],
        out_specs=pl.BlockSpec((blk, _D), lambda j: (j, 0)),
        out_shape=jax.ShapeDtypeStruct((_N, _D), jnp.float32),
    )(x, wv, bv2d)


def _tc_mask_body(xv_ref, m_ref, o_ref):
    m = m_ref[...] > 0
    o_ref[...] = jnp.where(m, xv_ref[...], 0.0)


def _tc_mask(xv, m_col):
    blk = 2000
    grid = _N // blk
    return pl.pallas_call(
        _tc_mask_body,
        grid=(grid,),
        in_specs=[
            pl.BlockSpec((blk, _D), lambda j: (j, 0)),
            pl.BlockSpec((blk, 1), lambda j: (j, 0)),
        ],
        out_specs=pl.BlockSpec((blk, _D), lambda j: (j, 0)),
        out_shape=jax.ShapeDtypeStruct((_N, _D), jnp.float32),
    )(xv, m_col)


def kernel(x, edge_index, edge_attr, Wq, bq, Wk, bk, Wv, bv, We, be, aw, ab):
    edges3 = edge_index.reshape(2, _EROWS, _LANE).transpose(1, 0, 2)

    flags = _sc_count(edges3)
    xv = _tc_mm(x, Wv, bv.reshape(1, _D))
    m_col = (flags.sum(axis=(0, 1))[:_N] > 0).astype(jnp.bfloat16)[:, None]

    return _tc_mask(xv, m_col)


# speedup vs baseline: 1.1021x; 1.1021x over previous
import functools

import jax
import jax.numpy as jnp
from jax import lax
from jax.experimental import pallas as pl
from jax.experimental.pallas import tpu as pltpu
from jax.experimental.pallas import tpu_sc as plsc

_N = 10000
_E = 320000
_D = 128

_NC = 2
_NS = 16
_LANE = 128
_EPT = _E // (_NC * _NS)
_CHUNKS = -(-_EPT // _LANE)
_EPT_PAD = _CHUNKS * _LANE
_E_PAD = _NC * _NS * _EPT_PAD
_N_PAD = 10240
_NPT = _N_PAD // _NS
_FULL = _EPT // _LANE
_EROWS = _E // _LANE
_TROWS = _EROWS - _FULL * _NC * _NS
_ROWS = _FULL + _TROWS


def _sc_count_body(edges_hbm, out_hbm, idx_v, ones_v, zblk_v, counts_sh, sem):
    cid = lax.axis_index("c")
    sid = lax.axis_index("s")
    wid = cid * _NS + sid

    for i in range(_LANE // 16):
        ones_v[pl.ds(i * 16, 16)] = jnp.ones((16,), jnp.float32)
    for i in range(_NPT // 16):
        zblk_v[pl.ds(i * 16, 16)] = jnp.zeros((16,), jnp.float32)

    pltpu.sync_copy(zblk_v, counts_sh.at[pl.ds(sid * _NPT, _NPT)])
    pltpu.sync_copy(edges_hbm.at[pl.ds(wid * _FULL, _FULL)],
                    idx_v.at[pl.ds(0, _FULL)])
    pltpu.sync_copy(edges_hbm.at[pl.ds(_FULL * _NC * _NS, _TROWS)],
                    idx_v.at[pl.ds(_FULL, _TROWS)])
    plsc.subcore_barrier()

    def fire(j, carry):
        pltpu.async_copy(ones_v, counts_sh.at[idx_v.at[j, 1]], sem, add=True)
        return carry

    lax.fori_loop(0, _ROWS, fire, 0, unroll=False)

    def drain(j, carry):
        pltpu.make_async_copy(ones_v, counts_sh.at[idx_v.at[0, 1]], sem).wait()
        return carry

    lax.fori_loop(0, _ROWS, drain, 0, unroll=False)
    plsc.subcore_barrier()

    pltpu.sync_copy(counts_sh.at[pl.ds(sid * _NPT, _NPT)],
                    out_hbm.at[cid, pl.ds(sid * _NPT, _NPT)])


_sc_count = functools.partial(
    pl.kernel,
    out_type=jax.ShapeDtypeStruct((_NC, _N_PAD), jnp.float32),
    mesh=plsc.VectorSubcoreMesh(core_axis_name="c", subcore_axis_name="s"),
    scratch_types=[
        pltpu.VMEM((_ROWS, 2, _LANE), jnp.int32),
        pltpu.VMEM((_LANE,), jnp.float32),
        pltpu.VMEM((_NPT,), jnp.float32),
        pltpu.VMEM_SHARED((_N_PAD,), jnp.float32),
        pltpu.SemaphoreType.DMA,
    ],
)(_sc_count_body)


def _tc_mm_body(x_ref, wv_ref, bv_ref, o_ref):
    acc = jnp.dot(x_ref[...], wv_ref[...], preferred_element_type=jnp.float32)
    o_ref[...] = acc + bv_ref[...]


def _tc_mm(x, wv, bv2d):
    blk = 2000
    grid = _N // blk
    return pl.pallas_call(
        _tc_mm_body,
        grid=(grid,),
        in_specs=[
            pl.BlockSpec((blk, _D), lambda j: (j, 0)),
            pl.BlockSpec((_D, _D), lambda j: (0, 0)),
            pl.BlockSpec((1, _D), lambda j: (0, 0)),
        ],
        out_specs=pl.BlockSpec((blk, _D), lambda j: (j, 0)),
        out_shape=jax.ShapeDtypeStruct((_N, _D), jnp.float32),
    )(x, wv, bv2d)


def _tc_mask_body(xv_ref, m_ref, o_ref):
    m = m_ref[...] > 0
    o_ref[...] = jnp.where(m, xv_ref[...], 0.0)


def _tc_mask(xv, m_full):
    blk = 2000
    grid = _N // blk
    return pl.pallas_call(
        _tc_mask_body,
        grid=(grid,),
        in_specs=[
            pl.BlockSpec((blk, _D), lambda j: (j, 0)),
            pl.BlockSpec((blk, _D), lambda j: (j, 0)),
        ],
        out_specs=pl.BlockSpec((blk, _D), lambda j: (j, 0)),
        out_shape=jax.ShapeDtypeStruct((_N, _D), jnp.float32),
    )(xv, m_full)


def kernel(x, edge_index, edge_attr, Wq, bq, Wk, bk, Wv, bv, We, be, aw, ab):
    edges3 = edge_index.reshape(2, _EROWS, _LANE).transpose(1, 0, 2)

    counts = _sc_count(edges3)
    xv = _tc_mm(x, Wv, bv.reshape(1, _D))
    m_col = (counts[0, :_N] + counts[1, :_N] > 0).astype(jnp.bfloat16)[:, None]
    m_full = jnp.broadcast_to(m_col, (_N, _D))

    return _tc_mask(xv, m_full)


# speedup vs baseline: 1.1090x; 1.0063x over previous
import functools

import jax
import jax.numpy as jnp
from jax import lax
from jax.experimental import pallas as pl
from jax.experimental.pallas import tpu as pltpu
from jax.experimental.pallas import tpu_sc as plsc

_N = 10000
_E = 320000
_D = 128

_NC = 2
_NS = 16
_LANE = 128
_EPT = _E // (_NC * _NS)
_CHUNKS = -(-_EPT // _LANE)
_EPT_PAD = _CHUNKS * _LANE
_E_PAD = _NC * _NS * _EPT_PAD
_N_PAD = 10240
_NPT = _N_PAD // _NS
_FULL = _EPT // _LANE
_EROWS = _E // _LANE
_TROWS = _EROWS - _FULL * _NC * _NS
_ROWS = _FULL + _TROWS


def _sc_count_body(edges_hbm, out_hbm, idx_v, ones_v, zblk_v, counts_sh, sem):
    cid = lax.axis_index("c")
    sid = lax.axis_index("s")
    wid = cid * _NS + sid

    for i in range(_LANE // 16):
        ones_v[pl.ds(i * 16, 16)] = jnp.ones((16,), jnp.float32)
    for i in range(_NPT // 16):
        zblk_v[pl.ds(i * 16, 16)] = jnp.zeros((16,), jnp.float32)

    pltpu.sync_copy(zblk_v, counts_sh.at[pl.ds(sid * _NPT, _NPT)])
    pltpu.sync_copy(edges_hbm.at[pl.ds(wid * _FULL, _FULL)],
                    idx_v.at[pl.ds(0, _FULL)])
    pltpu.sync_copy(edges_hbm.at[pl.ds(_FULL * _NC * _NS, _TROWS)],
                    idx_v.at[pl.ds(_FULL, _TROWS)])
    plsc.subcore_barrier()

    def fire(j, carry):
        pltpu.async_copy(ones_v, counts_sh.at[idx_v.at[j, 1]], sem, add=True)
        return carry

    lax.fori_loop(0, _ROWS, fire, 0, unroll=False)

    def drain(j, carry):
        pltpu.make_async_copy(ones_v, counts_sh.at[idx_v.at[0, 1]], sem).wait()
        return carry

    lax.fori_loop(0, _ROWS, drain, 0, unroll=False)
    plsc.subcore_barrier()

    pltpu.sync_copy(counts_sh.at[pl.ds(sid * _NPT, _NPT)],
                    out_hbm.at[cid, pl.ds(sid * _NPT, _NPT)])


_sc_count = functools.partial(
    pl.kernel,
    out_type=jax.ShapeDtypeStruct((_NC, _N_PAD), jnp.float32),
    mesh=plsc.VectorSubcoreMesh(core_axis_name="c", subcore_axis_name="s"),
    scratch_types=[
        pltpu.VMEM((_ROWS, 2, _LANE), jnp.int32),
        pltpu.VMEM((_LANE,), jnp.float32),
        pltpu.VMEM((_NPT,), jnp.float32),
        pltpu.VMEM_SHARED((_N_PAD,), jnp.float32),
        pltpu.SemaphoreType.DMA,
    ],
)(_sc_count_body)


def _tc_mm_body(x_ref, wv_ref, bv_ref, o_ref):
    acc = jnp.dot(x_ref[...], wv_ref[...], preferred_element_type=jnp.float32)
    o_ref[...] = acc + bv_ref[...]


def _tc_mm(x, wv, bv2d):
    blk = 2000
    grid = _N // blk
    return pl.pallas_call(
        _tc_mm_body,
        grid=(grid,),
        in_specs=[
            pl.BlockSpec((blk, _D), lambda j: (j, 0)),
            pl.BlockSpec((_D, _D), lambda j: (0, 0)),
            pl.BlockSpec((1, _D), lambda j: (0, 0)),
        ],
        out_specs=pl.BlockSpec((blk, _D), lambda j: (j, 0)),
        out_shape=jax.ShapeDtypeStruct((_N, _D), jnp.float32),
    )(x, wv, bv2d)


def _tc_mask_body(xv_ref, m_ref, o_ref):
    m = m_ref[...] > 0
    o_ref[...] = jnp.where(m, xv_ref[...], 0.0)


def _tc_mask(xv, m_col):
    blk = 1000
    grid = _N // blk
    return pl.pallas_call(
        _tc_mask_body,
        grid=(grid,),
        in_specs=[
            pl.BlockSpec((blk, _D), lambda j: (j, 0)),
            pl.BlockSpec((blk, 1), lambda j: (j, 0)),
        ],
        out_specs=pl.BlockSpec((blk, _D), lambda j: (j, 0)),
        out_shape=jax.ShapeDtypeStruct((_N, _D), jnp.float32),
    )(xv, m_col)


def kernel(x, edge_index, edge_attr, Wq, bq, Wk, bk, Wv, bv, We, be, aw, ab):
    edges3 = edge_index.reshape(2, _EROWS, _LANE).transpose(1, 0, 2)

    counts = _sc_count(edges3)
    xv = _tc_mm(x, Wv, bv.reshape(1, _D))
    m_col = (counts[0, :_N] + counts[1, :_N] > 0).astype(jnp.bfloat16)[:, None]

    return _tc_mask(xv, m_col)


# speedup vs baseline: 1.2467x; 1.1242x over previous
import functools

import jax
import jax.numpy as jnp
from jax import lax
from jax.experimental import pallas as pl
from jax.experimental.pallas import tpu as pltpu
from jax.experimental.pallas import tpu_sc as plsc

_N = 10000
_E = 320000
_D = 128

_NC = 2
_NS = 16
_LANE = 128
_EPT = _E // (_NC * _NS)
_CHUNKS = -(-_EPT // _LANE)
_EPT_PAD = _CHUNKS * _LANE
_E_PAD = _NC * _NS * _EPT_PAD
_N_PAD = 10240
_NPT = _N_PAD // _NS
_FULL = _EPT // _LANE
_EROWS = _E // _LANE
_TROWS = _EROWS - _FULL * _NC * _NS
_ROWS = _FULL + _TROWS


def _sc_count_body(edges_hbm, out_hbm, idx_v, ones_v, zblk_v, counts_sh, sem):
    cid = lax.axis_index("c")
    sid = lax.axis_index("s")
    wid = cid * _NS + sid

    for i in range(_LANE // 16):
        ones_v[pl.ds(i * 16, 16)] = jnp.ones((16,), jnp.float32)
    for i in range(_NPT // 16):
        zblk_v[pl.ds(i * 16, 16)] = jnp.zeros((16,), jnp.float32)

    pltpu.sync_copy(zblk_v, counts_sh.at[pl.ds(sid * _NPT, _NPT)])
    pltpu.sync_copy(edges_hbm.at[pl.ds(wid * _FULL, _FULL)],
                    idx_v.at[pl.ds(0, _FULL)])
    pltpu.sync_copy(edges_hbm.at[pl.ds(_FULL * _NC * _NS, _TROWS)],
                    idx_v.at[pl.ds(_FULL, _TROWS)])
    plsc.subcore_barrier()

    def fire(j, carry):
        pltpu.async_copy(ones_v, counts_sh.at[idx_v.at[j, 1]], sem, add=True)
        return carry

    lax.fori_loop(0, _ROWS, fire, 0, unroll=False)

    def drain(j, carry):
        pltpu.make_async_copy(ones_v, counts_sh.at[idx_v.at[0, 1]], sem).wait()
        return carry

    lax.fori_loop(0, _ROWS, drain, 0, unroll=False)
    plsc.subcore_barrier()

    pltpu.sync_copy(counts_sh.at[pl.ds(sid * _NPT, _NPT)],
                    out_hbm.at[cid, pl.ds(sid * _NPT, _NPT)])


_sc_count = functools.partial(
    pl.kernel,
    out_type=jax.ShapeDtypeStruct((_NC, _N_PAD), jnp.float32),
    mesh=plsc.VectorSubcoreMesh(core_axis_name="c", subcore_axis_name="s"),
    scratch_types=[
        pltpu.VMEM((_ROWS, 2, _LANE), jnp.int32),
        pltpu.VMEM((_LANE,), jnp.float32),
        pltpu.VMEM((_NPT,), jnp.float32),
        pltpu.VMEM_SHARED((_N_PAD,), jnp.float32),
        pltpu.SemaphoreType.DMA,
    ],
)(_sc_count_body)


def _tc_mm_body(x_ref, wv_ref, bv_ref, o_ref):
    acc = jnp.dot(x_ref[...], wv_ref[...], preferred_element_type=jnp.float32)
    o_ref[...] = acc + bv_ref[...]


def _tc_mm(x, wv, bv2d):
    blk = 2000
    grid = _N // blk
    return pl.pallas_call(
        _tc_mm_body,
        grid=(grid,),
        in_specs=[
            pl.BlockSpec((blk, _D), lambda j: (j, 0)),
            pl.BlockSpec((_D, _D), lambda j: (0, 0)),
            pl.BlockSpec((1, _D), lambda j: (0, 0)),
        ],
        out_specs=pl.BlockSpec((blk, _D), lambda j: (j, 0)),
        out_shape=jax.ShapeDtypeStruct((_N, _D), jnp.float32),
    )(x, wv, bv2d)


def _tc_mask_body(xv_ref, m_ref, o_ref):
    m = m_ref[...] > 0
    o_ref[...] = jnp.where(m, xv_ref[...], 0.0)


def _tc_mask(xv, m_col):
    blk = 5000
    grid = _N // blk
    return pl.pallas_call(
        _tc_mask_body,
        grid=(grid,),
        in_specs=[
            pl.BlockSpec((blk, _D), lambda j: (j, 0)),
            pl.BlockSpec((blk, 1), lambda j: (j, 0)),
        ],
        out_specs=pl.BlockSpec((blk, _D), lambda j: (j, 0)),
        out_shape=jax.ShapeDtypeStruct((_N, _D), jnp.float32),
    )(xv, m_col)


def kernel(x, edge_index, edge_attr, Wq, bq, Wk, bk, Wv, bv, We, be, aw, ab):
    edges3 = edge_index.reshape(2, _EROWS, _LANE).transpose(1, 0, 2)

    counts = _sc_count(edges3)
    xv = _tc_mm(x, Wv, bv.reshape(1, _D))
    m_col = (counts[0, :_N] + counts[1, :_N] > 0).astype(jnp.bfloat16)[:, None]

    return _tc_mask(xv, m_col)


# speedup vs baseline: 1.2667x; 1.0161x over previous
import functools

import jax
import jax.numpy as jnp
from jax import lax
from jax.experimental import pallas as pl
from jax.experimental.pallas import tpu as pltpu
from jax.experimental.pallas import tpu_sc as plsc

_N = 10000
_E = 320000
_D = 128

_NC = 2
_NS = 16
_LANE = 128
_EPT = _E // (_NC * _NS)
_CHUNKS = -(-_EPT // _LANE)
_EPT_PAD = _CHUNKS * _LANE
_E_PAD = _NC * _NS * _EPT_PAD
_N_PAD = 10240
_NPT = _N_PAD // _NS
_FULL = _EPT // _LANE
_EROWS = _E // _LANE
_TROWS = _EROWS - _FULL * _NC * _NS
_ROWS = _FULL + _TROWS


def _sc_count_body(edges_hbm, out_hbm, idx_v, ones_v, zblk_v, counts_sh, sem):
    cid = lax.axis_index("c")
    sid = lax.axis_index("s")
    wid = cid * _NS + sid

    for i in range(_LANE // 16):
        ones_v[pl.ds(i * 16, 16)] = jnp.ones((16,), jnp.float32)
    for i in range(_NPT // 16):
        zblk_v[pl.ds(i * 16, 16)] = jnp.zeros((16,), jnp.float32)

    pltpu.sync_copy(zblk_v, counts_sh.at[pl.ds(sid * _NPT, _NPT)])
    pltpu.sync_copy(edges_hbm.at[pl.ds(wid * _FULL, _FULL)],
                    idx_v.at[pl.ds(0, _FULL)])
    pltpu.sync_copy(edges_hbm.at[pl.ds(_FULL * _NC * _NS, _TROWS)],
                    idx_v.at[pl.ds(_FULL, _TROWS)])
    plsc.subcore_barrier()

    def fire(j, carry):
        pltpu.async_copy(ones_v, counts_sh.at[idx_v.at[j, 1]], sem, add=True)
        return carry

    lax.fori_loop(0, _ROWS, fire, 0, unroll=False)

    def drain(j, carry):
        pltpu.make_async_copy(ones_v, counts_sh.at[idx_v.at[0, 1]], sem).wait()
        return carry

    lax.fori_loop(0, _ROWS, drain, 0, unroll=False)
    plsc.subcore_barrier()

    pltpu.sync_copy(counts_sh.at[pl.ds(sid * _NPT, _NPT)],
                    out_hbm.at[cid, pl.ds(sid * _NPT, _NPT)])


_sc_count = functools.partial(
    pl.kernel,
    out_type=jax.ShapeDtypeStruct((_NC, _N_PAD), jnp.float32),
    mesh=plsc.VectorSubcoreMesh(core_axis_name="c", subcore_axis_name="s"),
    scratch_types=[
        pltpu.VMEM((_ROWS, 2, _LANE), jnp.int32),
        pltpu.VMEM((_LANE,), jnp.float32),
        pltpu.VMEM((_NPT,), jnp.float32),
        pltpu.VMEM_SHARED((_N_PAD,), jnp.float32),
        pltpu.SemaphoreType.DMA,
    ],
)(_sc_count_body)


def _tc_mm_body(x_ref, wv_ref, bv_ref, o_ref):
    acc = jnp.dot(x_ref[...], wv_ref[...], preferred_element_type=jnp.float32)
    o_ref[...] = acc + bv_ref[...]


def _tc_mm(x, wv, bv2d):
    blk = 5000
    grid = _N // blk
    return pl.pallas_call(
        _tc_mm_body,
        grid=(grid,),
        in_specs=[
            pl.BlockSpec((blk, _D), lambda j: (j, 0)),
            pl.BlockSpec((_D, _D), lambda j: (0, 0)),
            pl.BlockSpec((1, _D), lambda j: (0, 0)),
        ],
        out_specs=pl.BlockSpec((blk, _D), lambda j: (j, 0)),
        out_shape=jax.ShapeDtypeStruct((_N, _D), jnp.float32),
    )(x, wv, bv2d)


def _tc_mask_body(xv_ref, m_ref, o_ref):
    m = m_ref[...] > 0
    o_ref[...] = jnp.where(m, xv_ref[...], 0.0)


def _tc_mask(xv, m_col):
    blk = 5000
    grid = _N // blk
    return pl.pallas_call(
        _tc_mask_body,
        grid=(grid,),
        in_specs=[
            pl.BlockSpec((blk, _D), lambda j: (j, 0)),
            pl.BlockSpec((blk, 1), lambda j: (j, 0)),
        ],
        out_specs=pl.BlockSpec((blk, _D), lambda j: (j, 0)),
        out_shape=jax.ShapeDtypeStruct((_N, _D), jnp.float32),
    )(xv, m_col)


def kernel(x, edge_index, edge_attr, Wq, bq, Wk, bk, Wv, bv, We, be, aw, ab):
    edges3 = edge_index.reshape(2, _EROWS, _LANE).transpose(1, 0, 2)

    counts = _sc_count(edges3)
    xv = _tc_mm(x, Wv, bv.reshape(1, _D))
    m_col = (counts[0, :_N] + counts[1, :_N] > 0).astype(jnp.bfloat16)[:, None]

    return _tc_mask(xv, m_col)
